# Initial kernel scaffold; baseline (speedup 1.0000x reference)
import jax
import jax.numpy as jnp
from jax.experimental import pallas as pl


def kernel(x, edge_index, batch, W_embed, b_embed, W1s, b1s, g1s, be1s, W2s, b2s, W_task, b_task):
    raise NotImplementedError("write your pallas kernel here")


# speedup vs baseline: 2.6418x; 2.6418x over previous
import functools

import jax
import jax.numpy as jnp
from jax import lax
from jax.experimental import pallas as pl
from jax.experimental.pallas import tpu as pltpu
from jax.experimental.pallas import tpu_sc as plsc

N = 10000
D = 128
G = 64
NPAD = 10240
EROWS = 2560
NWORKERS = 32
ROWS_PER_W = EROWS // NWORKERS
RB = 1000
NBLK = N // RB


def _seg_sum_edges(h, src2d, dst2d, zeros_pad):
    mesh = plsc.VectorSubcoreMesh(core_axis_name="c", subcore_axis_name="s")

    @functools.partial(
        pl.kernel,
        out_type=jax.ShapeDtypeStruct((2 * N, D), jnp.float32),
        mesh=mesh,
        scratch_types=[
            pltpu.VMEM((ROWS_PER_W, 128), jnp.int32),
            pltpu.VMEM((ROWS_PER_W, 128), jnp.int32),
            pltpu.VMEM((128, D), jnp.float32),
            pltpu.VMEM_SHARED((NPAD, D), jnp.float32),
            pltpu.SemaphoreType.DMA,
        ],
    )
    def ksc(h_hbm, src_hbm, dst_hbm, z_hbm, out_hbm, src_v, dst_v, rows_v, agg_s, sem):
        c = lax.axis_index("c")
        s = lax.axis_index("s")
        wid = c * 16 + s
        row0 = wid * ROWS_PER_W
        pltpu.sync_copy(z_hbm.at[pl.ds(s * 640, 640)], agg_s.at[pl.ds(s * 640, 640)])
        pltpu.sync_copy(src_hbm.at[pl.ds(row0, ROWS_PER_W)], src_v)
        pltpu.sync_copy(dst_hbm.at[pl.ds(row0, ROWS_PER_W)], dst_v)
        plsc.subcore_barrier()

        def body(j, carry):
            pltpu.async_copy(h_hbm.at[src_v.at[j]], rows_v, sem).wait()
            pltpu.sync_copy(rows_v, agg_s.at[dst_v.at[j]], add=True)
            return carry

        lax.fori_loop(0, ROWS_PER_W, body, 0)
        plsc.subcore_barrier()
        off = s * 624
        pltpu.sync_copy(agg_s.at[pl.ds(off, 624)],
                        out_hbm.at[pl.ds(c * N + off, 624)])

        @pl.when(s == 0)
        def _():
            pltpu.sync_copy(agg_s.at[pl.ds(9984, 16)],
                            out_hbm.at[pl.ds(c * N + 9984, 16)])

    return ksc(h, src2d, dst2d, zeros_pad)


def _embed(x, W, b):
    def body(x_ref, w_ref, b_ref, o_ref):
        o_ref[...] = jnp.dot(x_ref[...], w_ref[...],
                             preferred_element_type=jnp.float32) + b_ref[...]

    return pl.pallas_call(
        body,
        grid=(NBLK,),
        in_specs=[
            pl.BlockSpec((RB, D), lambda i: (i, 0)),
            pl.BlockSpec((D, D), lambda i: (0, 0)),
            pl.BlockSpec((1, D), lambda i: (0, 0)),
        ],
        out_specs=pl.BlockSpec((RB, D), lambda i: (i, 0)),
        out_shape=jax.ShapeDtypeStruct((N, D), jnp.float32),
    )(x, W, b)


def _layer(h, parts, W1, b1, g1, be1, W2, b2, relu_out):
    def body(h_ref, p0_ref, p1_ref, w1_ref, b1_ref, g1_ref, be1_ref, w2_ref,
             b2_ref, o_ref, t_buf, sum_ref, ssq_ref):
        p = pl.program_id(0)
        i = pl.program_id(1)

        @pl.when(p == 0)
        def _():
            z = h_ref[...] + p0_ref[...] + p1_ref[...]
            t = jnp.dot(z, w1_ref[...],
                        preferred_element_type=jnp.float32) + b1_ref[...]
            t_buf[pl.ds(i * RB, RB), :] = t

            @pl.when(i == 0)
            def _():
                sum_ref[...] = jnp.zeros_like(sum_ref)
                ssq_ref[...] = jnp.zeros_like(ssq_ref)

            sum_ref[...] += jnp.sum(t, axis=0, keepdims=True)
            ssq_ref[...] += jnp.sum(t * t, axis=0, keepdims=True)

        @pl.when(p == 1)
        def _():
            mean = sum_ref[...] * (1.0 / N)
            var = ssq_ref[...] * (1.0 / N) - mean * mean
            inv = lax.rsqrt(var + 1e-5)
            t = t_buf[pl.ds(i * RB, RB), :]
            zn = (t - mean) * (inv * g1_ref[...]) + be1_ref[...]
            zn = jnp.maximum(zn, 0.0)
            o = jnp.dot(zn, w2_ref[...],
                        preferred_element_type=jnp.float32) + b2_ref[...]
            if relu_out:
                o = jnp.maximum(o, 0.0)
            o_ref[...] = o

    return pl.pallas_call(
        body,
        grid=(2, NBLK),
        in_specs=[
            pl.BlockSpec((RB, D), lambda p, i: (i, 0)),
            pl.BlockSpec((RB, D), lambda p, i: (i, 0)),
            pl.BlockSpec((RB, D), lambda p, i: (NBLK + i, 0)),
            pl.BlockSpec((D, D), lambda p, i: (0, 0)),
            pl.BlockSpec((1, D), lambda p, i: (0, 0)),
            pl.BlockSpec((1, D), lambda p, i: (0, 0)),
            pl.BlockSpec((1, D), lambda p, i: (0, 0)),
            pl.BlockSpec((D, D), lambda p, i: (0, 0)),
            pl.BlockSpec((1, D), lambda p, i: (0, 0)),
        ],
        out_specs=pl.BlockSpec((RB, D), lambda p, i: (i, 0)),
        out_shape=jax.ShapeDtypeStruct((N, D), jnp.float32),
        scratch_shapes=[
            pltpu.VMEM((N, D), jnp.float32),
            pltpu.VMEM((1, D), jnp.float32),
            pltpu.VMEM((1, D), jnp.float32),
        ],
    )(h, parts, parts, W1, b1, g1, be1, W2, b2)


def _pool_task(h, batch2d, Wt, bt):
    def body(h_ref, b_ref, wt_ref, bt_ref, o_ref, acc_ref):
        i = pl.program_id(0)

        @pl.when(i == 0)
        def _():
            acc_ref[...] = jnp.zeros_like(acc_ref)

        onehot = (b_ref[...] == lax.broadcasted_iota(jnp.int32, (RB, G), 1)
                  ).astype(jnp.float32)
        acc_ref[...] += lax.dot_general(onehot, h_ref[...],
                                        (((0,), (0,)), ((), ())),
                                        preferred_element_type=jnp.float32)

        @pl.when(i == NBLK - 1)
        def _():
            o_ref[...] = jnp.dot(acc_ref[...], wt_ref[...],
                                 preferred_element_type=jnp.float32) + bt_ref[...]

    return pl.pallas_call(
        body,
        grid=(NBLK,),
        in_specs=[
            pl.BlockSpec((RB, D), lambda i: (i, 0)),
            pl.BlockSpec((RB, 1), lambda i: (i, 0)),
            pl.BlockSpec((D, 1), lambda i: (0, 0)),
            pl.BlockSpec((1, 1), lambda i: (0, 0)),
        ],
        out_specs=pl.BlockSpec((G, 1), lambda i: (0, 0)),
        out_shape=jax.ShapeDtypeStruct((G, 1), jnp.float32),
        scratch_shapes=[pltpu.VMEM((G, D), jnp.float32)],
    )(h, batch2d, Wt, bt)


def kernel(x, edge_index, batch, W_embed, b_embed, W1s, b1s, g1s, be1s, W2s,
           b2s, W_task, b_task):
    e = edge_index.astype(jnp.int32)
    pad = EROWS * 128 - e.shape[1]
    src2d = jnp.concatenate([e[0], jnp.zeros((pad,), jnp.int32)]).reshape(EROWS, 128)
    dst2d = jnp.concatenate([e[1], jnp.full((pad,), N, jnp.int32)]).reshape(EROWS, 128)
    zeros_pad = jnp.zeros((NPAD, D), jnp.float32)
    batch2d = batch.astype(jnp.int32).reshape(N, 1)

    h = _embed(x, W_embed, b_embed.reshape(1, D))
    n_layers = W1s.shape[0]
    for l in range(n_layers):
        parts = _seg_sum_edges(h, src2d, dst2d, zeros_pad)
        h = _layer(h, parts, W1s[l], b1s[l].reshape(1, D), g1s[l].reshape(1, D),
                   be1s[l].reshape(1, D), W2s[l], b2s[l].reshape(1, D),
                   relu_out=(l < n_layers - 1))
    return _pool_task(h, batch2d, W_task.reshape(D, 1), b_task.reshape(1, 1))


# speedup vs baseline: 2.6697x; 1.0106x over previous
import functools

import jax
import jax.numpy as jnp
from jax import lax
from jax.experimental import pallas as pl
from jax.experimental.pallas import tpu as pltpu
from jax.experimental.pallas import tpu_sc as plsc

N = 10000
D = 128
G = 64
NPAD = 10240
EROWS = 2560
NWORKERS = 32
ROWS_PER_W = EROWS // NWORKERS
NBUF = 2
GIDX = 8
RB = 1000
NBLK = N // RB


def _seg_sum_edges(h, src2d, dst2d, zeros_pad):
    mesh = plsc.VectorSubcoreMesh(core_axis_name="c", subcore_axis_name="s")

    @functools.partial(
        pl.kernel,
        out_type=jax.ShapeDtypeStruct((2 * N, D), jnp.float32),
        mesh=mesh,
        scratch_types=[
            pltpu.VMEM((GIDX, 128), jnp.int32),
            pltpu.VMEM((GIDX, 128), jnp.int32),
            pltpu.VMEM((NBUF, 128, D), jnp.float32),
            pltpu.VMEM_SHARED((NPAD, D), jnp.float32),
            pltpu.SemaphoreType.DMA((NBUF,)),
        ],
    )
    def ksc(h_hbm, src_hbm, dst_hbm, z_hbm, out_hbm, src_v, dst_v, rows_b, agg_s,
            gsem):
        rows_v = [rows_b.at[b] for b in range(NBUF)]
        c = lax.axis_index("c")
        s = lax.axis_index("s")
        wid = c * 16 + s
        row0 = wid * ROWS_PER_W
        pltpu.sync_copy(z_hbm.at[pl.ds(s * 640, 640)], agg_s.at[pl.ds(s * 640, 640)])
        plsc.subcore_barrier()

        def group(g, carry):
            base = row0 + g * GIDX
            pltpu.sync_copy(src_hbm.at[pl.ds(base, GIDX)], src_v)
            pltpu.sync_copy(dst_hbm.at[pl.ds(base, GIDX)], dst_v)

            def inner(k, carry2):
                handles = [
                    pltpu.async_copy(h_hbm.at[src_v.at[k * NBUF + b]],
                                     rows_v[b], gsem.at[b])
                    for b in range(NBUF)
                ]
                for b in range(NBUF):
                    handles[b].wait()
                    pltpu.sync_copy(rows_v[b],
                                    agg_s.at[dst_v.at[k * NBUF + b]], add=True)
                return carry2

            lax.fori_loop(0, GIDX // NBUF, inner, 0)
            return carry

        lax.fori_loop(0, ROWS_PER_W // GIDX, group, 0)
        plsc.subcore_barrier()
        off = s * 624
        pltpu.sync_copy(agg_s.at[pl.ds(off, 624)],
                        out_hbm.at[pl.ds(c * N + off, 624)])

        @pl.when(s == 0)
        def _():
            pltpu.sync_copy(agg_s.at[pl.ds(9984, 16)],
                            out_hbm.at[pl.ds(c * N + 9984, 16)])

    return ksc(h, src2d, dst2d, zeros_pad)


def _embed(x, W, b):
    def body(x_ref, w_ref, b_ref, o_ref):
        o_ref[...] = jnp.dot(x_ref[...], w_ref[...],
                             preferred_element_type=jnp.float32) + b_ref[...]

    return pl.pallas_call(
        body,
        grid=(NBLK,),
        in_specs=[
            pl.BlockSpec((RB, D), lambda i: (i, 0)),
            pl.BlockSpec((D, D), lambda i: (0, 0)),
            pl.BlockSpec((1, D), lambda i: (0, 0)),
        ],
        out_specs=pl.BlockSpec((RB, D), lambda i: (i, 0)),
        out_shape=jax.ShapeDtypeStruct((N, D), jnp.float32),
    )(x, W, b)


def _layer(h, parts, W1, b1, g1, be1, W2, b2, relu_out):
    def body(h_ref, p0_ref, p1_ref, w1_ref, b1_ref, g1_ref, be1_ref, w2_ref,
             b2_ref, o_ref, t_buf, sum_ref, ssq_ref):
        p = pl.program_id(0)
        i = pl.program_id(1)

        @pl.when(p == 0)
        def _():
            z = h_ref[...] + p0_ref[...] + p1_ref[...]
            t = jnp.dot(z, w1_ref[...],
                        preferred_element_type=jnp.float32) + b1_ref[...]
            t_buf[pl.ds(i * RB, RB), :] = t

            @pl.when(i == 0)
            def _():
                sum_ref[...] = jnp.zeros_like(sum_ref)
                ssq_ref[...] = jnp.zeros_like(ssq_ref)

            sum_ref[...] += jnp.sum(t, axis=0, keepdims=True)
            ssq_ref[...] += jnp.sum(t * t, axis=0, keepdims=True)

        @pl.when(p == 1)
        def _():
            mean = sum_ref[...] * (1.0 / N)
            var = ssq_ref[...] * (1.0 / N) - mean * mean
            inv = lax.rsqrt(var + 1e-5)
            t = t_buf[pl.ds(i * RB, RB), :]
            zn = (t - mean) * (inv * g1_ref[...]) + be1_ref[...]
            zn = jnp.maximum(zn, 0.0)
            o = jnp.dot(zn, w2_ref[...],
                        preferred_element_type=jnp.float32) + b2_ref[...]
            if relu_out:
                o = jnp.maximum(o, 0.0)
            o_ref[...] = o

    return pl.pallas_call(
        body,
        grid=(2, NBLK),
        in_specs=[
            pl.BlockSpec((RB, D), lambda p, i: (i, 0)),
            pl.BlockSpec((RB, D), lambda p, i: (i, 0)),
            pl.BlockSpec((RB, D), lambda p, i: (NBLK + i, 0)),
            pl.BlockSpec((D, D), lambda p, i: (0, 0)),
            pl.BlockSpec((1, D), lambda p, i: (0, 0)),
            pl.BlockSpec((1, D), lambda p, i: (0, 0)),
            pl.BlockSpec((1, D), lambda p, i: (0, 0)),
            pl.BlockSpec((D, D), lambda p, i: (0, 0)),
            pl.BlockSpec((1, D), lambda p, i: (0, 0)),
        ],
        out_specs=pl.BlockSpec((RB, D), lambda p, i: (p * i, 0)),
        out_shape=jax.ShapeDtypeStruct((N, D), jnp.float32),
        scratch_shapes=[
            pltpu.VMEM((N, D), jnp.float32),
            pltpu.VMEM((1, D), jnp.float32),
            pltpu.VMEM((1, D), jnp.float32),
        ],
    )(h, parts, parts, W1, b1, g1, be1, W2, b2)


def _last_layer_pool(h, parts, W1, b1, g1, be1, W2, b2, batch2d, Wt, bt):
    def body(h_ref, p0_ref, p1_ref, w1_ref, b1_ref, g1_ref, be1_ref, w2_ref,
             b2_ref, bat_ref, wt_ref, bt_ref, o_ref, t_buf, sum_ref, ssq_ref,
             acc_ref):
        p = pl.program_id(0)
        i = pl.program_id(1)

        @pl.when(p == 0)
        def _():
            z = h_ref[...] + p0_ref[...] + p1_ref[...]
            t = jnp.dot(z, w1_ref[...],
                        preferred_element_type=jnp.float32) + b1_ref[...]
            t_buf[pl.ds(i * RB, RB), :] = t

            @pl.when(i == 0)
            def _():
                sum_ref[...] = jnp.zeros_like(sum_ref)
                ssq_ref[...] = jnp.zeros_like(ssq_ref)

            sum_ref[...] += jnp.sum(t, axis=0, keepdims=True)
            ssq_ref[...] += jnp.sum(t * t, axis=0, keepdims=True)

        @pl.when(p == 1)
        def _():
            mean = sum_ref[...] * (1.0 / N)
            var = ssq_ref[...] * (1.0 / N) - mean * mean
            inv = lax.rsqrt(var + 1e-5)
            t = t_buf[pl.ds(i * RB, RB), :]
            zn = (t - mean) * (inv * g1_ref[...]) + be1_ref[...]
            zn = jnp.maximum(zn, 0.0)
            o = jnp.dot(zn, w2_ref[...],
                        preferred_element_type=jnp.float32) + b2_ref[...]

            @pl.when(i == 0)
            def _():
                acc_ref[...] = jnp.zeros_like(acc_ref)

            onehot = (bat_ref[...] == lax.broadcasted_iota(jnp.int32, (RB, G), 1)
                      ).astype(jnp.float32)
            acc_ref[...] += lax.dot_general(onehot, o, (((0,), (0,)), ((), ())),
                                            preferred_element_type=jnp.float32)

            @pl.when(i == NBLK - 1)
            def _():
                o_ref[...] = jnp.dot(acc_ref[...], wt_ref[...],
                                     preferred_element_type=jnp.float32) + bt_ref[...]

    return pl.pallas_call(
        body,
        grid=(2, NBLK),
        in_specs=[
            pl.BlockSpec((RB, D), lambda p, i: (i, 0)),
            pl.BlockSpec((RB, D), lambda p, i: (i, 0)),
            pl.BlockSpec((RB, D), lambda p, i: (NBLK + i, 0)),
            pl.BlockSpec((D, D), lambda p, i: (0, 0)),
            pl.BlockSpec((1, D), lambda p, i: (0, 0)),
            pl.BlockSpec((1, D), lambda p, i: (0, 0)),
            pl.BlockSpec((1, D), lambda p, i: (0, 0)),
            pl.BlockSpec((D, D), lambda p, i: (0, 0)),
            pl.BlockSpec((1, D), lambda p, i: (0, 0)),
            pl.BlockSpec((RB, 1), lambda p, i: (i, 0)),
            pl.BlockSpec((D, 1), lambda p, i: (0, 0)),
            pl.BlockSpec((1, 1), lambda p, i: (0, 0)),
        ],
        out_specs=pl.BlockSpec((G, 1), lambda p, i: (0, 0)),
        out_shape=jax.ShapeDtypeStruct((G, 1), jnp.float32),
        scratch_shapes=[
            pltpu.VMEM((N, D), jnp.float32),
            pltpu.VMEM((1, D), jnp.float32),
            pltpu.VMEM((1, D), jnp.float32),
            pltpu.VMEM((G, D), jnp.float32),
        ],
    )(h, parts, parts, W1, b1, g1, be1, W2, b2, batch2d, Wt, bt)


def _pool_task(h, batch2d, Wt, bt):
    def body(h_ref, b_ref, wt_ref, bt_ref, o_ref, acc_ref):
        i = pl.program_id(0)

        @pl.when(i == 0)
        def _():
            acc_ref[...] = jnp.zeros_like(acc_ref)

        onehot = (b_ref[...] == lax.broadcasted_iota(jnp.int32, (RB, G), 1)
                  ).astype(jnp.float32)
        acc_ref[...] += lax.dot_general(onehot, h_ref[...],
                                        (((0,), (0,)), ((), ())),
                                        preferred_element_type=jnp.float32)

        @pl.when(i == NBLK - 1)
        def _():
            o_ref[...] = jnp.dot(acc_ref[...], wt_ref[...],
                                 preferred_element_type=jnp.float32) + bt_ref[...]

    return pl.pallas_call(
        body,
        grid=(NBLK,),
        in_specs=[
            pl.BlockSpec((RB, D), lambda i: (i, 0)),
            pl.BlockSpec((RB, 1), lambda i: (i, 0)),
            pl.BlockSpec((D, 1), lambda i: (0, 0)),
            pl.BlockSpec((1, 1), lambda i: (0, 0)),
        ],
        out_specs=pl.BlockSpec((G, 1), lambda i: (0, 0)),
        out_shape=jax.ShapeDtypeStruct((G, 1), jnp.float32),
        scratch_shapes=[pltpu.VMEM((G, D), jnp.float32)],
    )(h, batch2d, Wt, bt)


def kernel(x, edge_index, batch, W_embed, b_embed, W1s, b1s, g1s, be1s, W2s,
           b2s, W_task, b_task):
    e = edge_index.astype(jnp.int32)
    pad = EROWS * 128 - e.shape[1]
    src2d = jnp.concatenate([e[0], jnp.zeros((pad,), jnp.int32)]).reshape(EROWS, 128)
    dst2d = jnp.concatenate([e[1], jnp.full((pad,), N, jnp.int32)]).reshape(EROWS, 128)
    zeros_pad = jnp.zeros((NPAD, D), jnp.float32)
    batch2d = batch.astype(jnp.int32).reshape(N, 1)

    h = _embed(x, W_embed, b_embed.reshape(1, D))
    n_layers = W1s.shape[0]
    for l in range(n_layers - 1):
        parts = _seg_sum_edges(h, src2d, dst2d, zeros_pad)
        h = _layer(h, parts, W1s[l], b1s[l].reshape(1, D), g1s[l].reshape(1, D),
                   be1s[l].reshape(1, D), W2s[l], b2s[l].reshape(1, D),
                   relu_out=True)
    ll = n_layers - 1
    parts = _seg_sum_edges(h, src2d, dst2d, zeros_pad)
    return _last_layer_pool(h, parts, W1s[ll], b1s[ll].reshape(1, D),
                            g1s[ll].reshape(1, D), be1s[ll].reshape(1, D),
                            W2s[ll], b2s[ll].reshape(1, D), batch2d,
                            W_task.reshape(D, 1), b_task.reshape(1, 1))


# speedup vs baseline: 3.1243x; 1.1703x over previous
import functools

import jax
import jax.numpy as jnp
from jax import lax
from jax.experimental import pallas as pl
from jax.experimental.pallas import tpu as pltpu
from jax.experimental.pallas import tpu_sc as plsc

N = 10000
D = 128
G = 64
NPAD = 10240
EROWS = 2560
NBUF = 2
GIDX = 8
ROWS_C0 = 120
ROWS_C1 = 40
RB = 1000
NBLK = N // RB


def _seg_sum_edges(h, src2d, dst2d, zeros_pad):
    mesh = plsc.VectorSubcoreMesh(core_axis_name="c", subcore_axis_name="s")

    @functools.partial(
        pl.kernel,
        out_type=jax.ShapeDtypeStruct((2 * N, D), jnp.float32),
        mesh=mesh,
        scratch_types=[
            pltpu.VMEM((GIDX, 128), jnp.int32),
            pltpu.VMEM((GIDX, 128), jnp.int32),
            pltpu.VMEM((NBUF, 128, D), jnp.float32),
            pltpu.VMEM_SHARED((NPAD, D), jnp.float32),
            pltpu.SemaphoreType.DMA((NBUF,)),
        ],
    )
    def ksc(h_hbm, src_hbm, dst_hbm, z_hbm, out_hbm, src_v, dst_v, rows_b, agg_s,
            gsem):
        rows_v = [rows_b.at[b] for b in range(NBUF)]
        c = lax.axis_index("c")
        s = lax.axis_index("s")
        row0 = jnp.where(c == 0, s * ROWS_C0, 16 * ROWS_C0 + s * ROWS_C1)
        ngroups = jnp.where(c == 0, ROWS_C0 // GIDX, ROWS_C1 // GIDX)
        pltpu.sync_copy(z_hbm.at[pl.ds(s * 640, 640)], agg_s.at[pl.ds(s * 640, 640)])
        plsc.subcore_barrier()

        def group(g, carry):
            base = row0 + g * GIDX
            pltpu.sync_copy(src_hbm.at[pl.ds(base, GIDX)], src_v)
            pltpu.sync_copy(dst_hbm.at[pl.ds(base, GIDX)], dst_v)

            def inner(k, carry2):
                handles = [
                    pltpu.async_copy(h_hbm.at[src_v.at[k * NBUF + b]],
                                     rows_v[b], gsem.at[b])
                    for b in range(NBUF)
                ]
                for b in range(NBUF):
                    handles[b].wait()
                    pltpu.sync_copy(rows_v[b],
                                    agg_s.at[dst_v.at[k * NBUF + b]], add=True)
                return carry2

            lax.fori_loop(0, GIDX // NBUF, inner, 0)
            return carry

        lax.fori_loop(0, ngroups, group, 0)
        plsc.subcore_barrier()
        off = s * 624
        pltpu.sync_copy(agg_s.at[pl.ds(off, 624)],
                        out_hbm.at[pl.ds(c * N + off, 624)])

        @pl.when(s == 0)
        def _():
            pltpu.sync_copy(agg_s.at[pl.ds(9984, 16)],
                            out_hbm.at[pl.ds(c * N + 9984, 16)])

    return ksc(h, src2d, dst2d, zeros_pad)


def _embed(x, W, b):
    def body(x_ref, w_ref, b_ref, o_ref):
        o_ref[...] = jnp.dot(x_ref[...], w_ref[...],
                             preferred_element_type=jnp.float32) + b_ref[...]

    return pl.pallas_call(
        body,
        grid=(NBLK,),
        in_specs=[
            pl.BlockSpec((RB, D), lambda i: (i, 0)),
            pl.BlockSpec((D, D), lambda i: (0, 0)),
            pl.BlockSpec((1, D), lambda i: (0, 0)),
        ],
        out_specs=pl.BlockSpec((RB, D), lambda i: (i, 0)),
        out_shape=jax.ShapeDtypeStruct((N, D), jnp.float32),
    )(x, W, b)


def _layer(h, parts, W1, b1, g1, be1, W2, b2):
    def body(h_ref, p0_ref, p1_ref, w1_ref, b1_ref, g1_ref, be1_ref, w2_ref,
             b2_ref, o_ref, t_buf, sum_ref, ssq_ref):
        p = pl.program_id(0)
        i = pl.program_id(1)

        @pl.when(p == 0)
        def _():
            z = h_ref[...] + p0_ref[...] + p1_ref[...]
            t = jnp.dot(z, w1_ref[...],
                        preferred_element_type=jnp.float32) + b1_ref[...]
            t_buf[pl.ds(i * RB, RB), :] = t

            @pl.when(i == 0)
            def _():
                sum_ref[...] = jnp.zeros_like(sum_ref)
                ssq_ref[...] = jnp.zeros_like(ssq_ref)

            sum_ref[...] += jnp.sum(t, axis=0, keepdims=True)
            ssq_ref[...] += jnp.sum(t * t, axis=0, keepdims=True)

        @pl.when(p == 1)
        def _():
            mean = sum_ref[...] * (1.0 / N)
            var = ssq_ref[...] * (1.0 / N) - mean * mean
            inv = 1.0 / jnp.sqrt(var + 1e-5)
            t = t_buf[pl.ds(i * RB, RB), :]
            zn = (t - mean) * (inv * g1_ref[...]) + be1_ref[...]
            zn = jnp.maximum(zn, 0.0)
            o = jnp.dot(zn, w2_ref[...],
                        preferred_element_type=jnp.float32) + b2_ref[...]
            o = jnp.maximum(o, 0.0)
            o_ref[...] = o

    return pl.pallas_call(
        body,
        grid=(2, NBLK),
        in_specs=[
            pl.BlockSpec((RB, D), lambda p, i: (i, 0)),
            pl.BlockSpec((RB, D), lambda p, i: (i, 0)),
            pl.BlockSpec((RB, D), lambda p, i: (NBLK + i, 0)),
            pl.BlockSpec((D, D), lambda p, i: (0, 0)),
            pl.BlockSpec((1, D), lambda p, i: (0, 0)),
            pl.BlockSpec((1, D), lambda p, i: (0, 0)),
            pl.BlockSpec((1, D), lambda p, i: (0, 0)),
            pl.BlockSpec((D, D), lambda p, i: (0, 0)),
            pl.BlockSpec((1, D), lambda p, i: (0, 0)),
        ],
        out_specs=pl.BlockSpec((RB, D), lambda p, i: (p * i, 0)),
        out_shape=jax.ShapeDtypeStruct((N, D), jnp.float32),
        scratch_shapes=[
            pltpu.VMEM((N, D), jnp.float32),
            pltpu.VMEM((1, D), jnp.float32),
            pltpu.VMEM((1, D), jnp.float32),
        ],
    )(h, parts, parts, W1, b1, g1, be1, W2, b2)


def _last_layer_pool(h, parts, W1, b1, g1, be1, W2, b2, batch2d, Wt, bt):
    def body(h_ref, p0_ref, p1_ref, w1_ref, b1_ref, g1_ref, be1_ref, w2_ref,
             b2_ref, bat_ref, wt_ref, bt_ref, o_ref, t_buf, sum_ref, ssq_ref,
             acc_ref):
        p = pl.program_id(0)
        i = pl.program_id(1)

        @pl.when(p == 0)
        def _():
            z = h_ref[...] + p0_ref[...] + p1_ref[...]
            t = jnp.dot(z, w1_ref[...],
                        preferred_element_type=jnp.float32) + b1_ref[...]
            t_buf[pl.ds(i * RB, RB), :] = t

            @pl.when(i == 0)
            def _():
                sum_ref[...] = jnp.zeros_like(sum_ref)
                ssq_ref[...] = jnp.zeros_like(ssq_ref)

            sum_ref[...] += jnp.sum(t, axis=0, keepdims=True)
            ssq_ref[...] += jnp.sum(t * t, axis=0, keepdims=True)

        @pl.when(p == 1)
        def _():
            mean = sum_ref[...] * (1.0 / N)
            var = ssq_ref[...] * (1.0 / N) - mean * mean
            inv = 1.0 / jnp.sqrt(var + 1e-5)
            t = t_buf[pl.ds(i * RB, RB), :]
            zn = (t - mean) * (inv * g1_ref[...]) + be1_ref[...]
            zn = jnp.maximum(zn, 0.0)
            o = jnp.dot(zn, w2_ref[...],
                        preferred_element_type=jnp.float32) + b2_ref[...]

            @pl.when(i == 0)
            def _():
                acc_ref[...] = jnp.zeros_like(acc_ref)

            onehot = (bat_ref[...] == lax.broadcasted_iota(jnp.int32, (RB, G), 1)
                      ).astype(jnp.float32)
            acc_ref[...] += lax.dot_general(onehot, o, (((0,), (0,)), ((), ())),
                                            preferred_element_type=jnp.float32,
                                            precision=lax.Precision.HIGHEST)

            @pl.when(i == NBLK - 1)
            def _():
                o_ref[...] = jnp.dot(acc_ref[...], wt_ref[...],
                                     preferred_element_type=jnp.float32) + bt_ref[...]

    return pl.pallas_call(
        body,
        grid=(2, NBLK),
        in_specs=[
            pl.BlockSpec((RB, D), lambda p, i: (i, 0)),
            pl.BlockSpec((RB, D), lambda p, i: (i, 0)),
            pl.BlockSpec((RB, D), lambda p, i: (NBLK + i, 0)),
            pl.BlockSpec((D, D), lambda p, i: (0, 0)),
            pl.BlockSpec((1, D), lambda p, i: (0, 0)),
            pl.BlockSpec((1, D), lambda p, i: (0, 0)),
            pl.BlockSpec((1, D), lambda p, i: (0, 0)),
            pl.BlockSpec((D, D), lambda p, i: (0, 0)),
            pl.BlockSpec((1, D), lambda p, i: (0, 0)),
            pl.BlockSpec((RB, 1), lambda p, i: (i, 0)),
            pl.BlockSpec((D, 1), lambda p, i: (0, 0)),
            pl.BlockSpec((1, 1), lambda p, i: (0, 0)),
        ],
        out_specs=pl.BlockSpec((G, 1), lambda p, i: (0, 0)),
        out_shape=jax.ShapeDtypeStruct((G, 1), jnp.float32),
        scratch_shapes=[
            pltpu.VMEM((N, D), jnp.float32),
            pltpu.VMEM((1, D), jnp.float32),
            pltpu.VMEM((1, D), jnp.float32),
            pltpu.VMEM((G, D), jnp.float32),
        ],
    )(h, parts, parts, W1, b1, g1, be1, W2, b2, batch2d, Wt, bt)


def kernel(x, edge_index, batch, W_embed, b_embed, W1s, b1s, g1s, be1s, W2s,
           b2s, W_task, b_task):
    e = edge_index.astype(jnp.int32)
    pad = EROWS * 128 - e.shape[1]
    src2d = jnp.concatenate([e[0], jnp.zeros((pad,), jnp.int32)]).reshape(EROWS, 128)
    dst2d = jnp.concatenate([e[1], jnp.full((pad,), N, jnp.int32)]).reshape(EROWS, 128)
    zeros_pad = jnp.zeros((NPAD, D), jnp.float32)
    batch2d = batch.astype(jnp.int32).reshape(N, 1)

    h = _embed(x, W_embed, b_embed.reshape(1, D))
    n_layers = W1s.shape[0]
    for l in range(n_layers - 1):
        parts = _seg_sum_edges(h, src2d, dst2d, zeros_pad)
        h = _layer(h, parts, W1s[l], b1s[l].reshape(1, D), g1s[l].reshape(1, D),
                   be1s[l].reshape(1, D), W2s[l], b2s[l].reshape(1, D))
    ll = n_layers - 1
    parts = _seg_sum_edges(h, src2d, dst2d, zeros_pad)
    return _last_layer_pool(h, parts, W1s[ll], b1s[ll].reshape(1, D),
                            g1s[ll].reshape(1, D), be1s[ll].reshape(1, D),
                            W2s[ll], b2s[ll].reshape(1, D), batch2d,
                            W_task.reshape(D, 1), b_task.reshape(1, 1))
